# Initial kernel scaffold; baseline (speedup 1.0000x reference)
import jax
import jax.numpy as jnp
from jax.experimental import pallas as pl


def kernel(x1, x2, ws_w, ws_b, msn_w, msn_b, wm_w, wm_b):
    raise NotImplementedError("write your pallas kernel here")


# speedup vs baseline: 41.0133x; 41.0133x over previous
import jax
import jax.numpy as jnp
from jax import lax
from jax.experimental import pallas as pl
from jax.experimental.pallas import tpu as pltpu

_WAY = 5
_SHOT = 5
_K = 3


def _top3sum(x):
    n = x.shape[1]
    iota = lax.broadcasted_iota(jnp.int32, x.shape, 1)
    total = jnp.zeros((x.shape[0], 1), jnp.float32)
    for k in range(_K):
        m = jnp.max(x, axis=1, keepdims=True)
        total = total + m
        if k < _K - 1:
            eq = x == m
            first = jnp.min(jnp.where(eq, iota, n), axis=1, keepdims=True)
            x = jnp.where(iota == first, -jnp.inf, x)
    return total


def _body(q_ref, s_ref, wsw_ref, par_ref, out_ref):
    C, U = s_ref.shape
    HW = q_ref.shape[1]
    SHW = _SHOT * HW
    ws_b = par_ref[0, 0]
    msn_w = par_ref[0, 1]
    msn_b = par_ref[0, 2]
    wm_w = par_ref[0, 3]
    wm_b = par_ref[0, 4]

    s = s_ref[...]
    sn = s * lax.rsqrt(jnp.sum(s * s, axis=0, keepdims=True))
    pw = jnp.zeros((C, 1), jnp.float32)
    for v in range(_WAY):
        pv = jnp.mean(sn[:, v * SHW:(v + 1) * SHW], axis=1, keepdims=True)
        pw = pw + wsw_ref[0, v] * pv
    ssw = jax.nn.sigmoid(jnp.sum(pw * sn, axis=0, keepdims=True) + ws_b)

    q = q_ref[0]
    qn = q * lax.rsqrt(jnp.sum(q * q, axis=1, keepdims=True))
    A = jnp.dot(qn, sn, preferred_element_type=jnp.float32)

    outs = []
    for v in range(_WAY):
        Av = A[:, v * SHW:(v + 1) * SHW]
        M = Av[:, 0:HW]
        for b in range(1, _SHOT):
            M = jnp.maximum(M, Av[:, b * HW:(b + 1) * HW])
        rel = _top3sum(msn_w * M + msn_b)
        w = jax.nn.sigmoid(wm_w * rel + wm_b)
        r2 = _top3sum(Av * ssw[:, v * SHW:(v + 1) * SHW])
        outs.append(jnp.sum(w * r2).reshape(1, 1, 1))
    out_ref[...] = jnp.concatenate(outs, axis=2)


@jax.jit
def kernel(x1, x2, ws_w, ws_b, msn_w, msn_b, wm_w, wm_b):
    Q, C, H, W = x1.shape
    HW = H * W
    S = x2.shape[0]
    U = S * HW

    x1t = x1.reshape(Q, C, HW).transpose(0, 2, 1)
    s_cols = x2.reshape(S, C, HW).transpose(1, 0, 2).reshape(C, U)
    s_perm = (s_cols.reshape(C, _WAY, HW, _SHOT)
              .transpose(0, 1, 3, 2).reshape(C, U))
    wsw = jnp.asarray(ws_w, jnp.float32).reshape(1, _WAY)
    par = jnp.stack([ws_b, msn_w, msn_b, wm_w, wm_b]).astype(
        jnp.float32).reshape(1, 5)

    out = pl.pallas_call(
        _body,
        grid=(Q,),
        in_specs=[
            pl.BlockSpec((1, HW, C), lambda i: (i, 0, 0)),
            pl.BlockSpec((C, U), lambda i: (0, 0)),
            pl.BlockSpec((1, _WAY), lambda i: (0, 0)),
            pl.BlockSpec((1, 5), lambda i: (0, 0)),
        ],
        out_specs=pl.BlockSpec((1, 1, _WAY), lambda i: (i, 0, 0)),
        out_shape=jax.ShapeDtypeStruct((Q, 1, _WAY), jnp.float32),
        compiler_params=pltpu.CompilerParams(
            dimension_semantics=("arbitrary",)),
    )(x1t, s_perm, wsw, par)
    return out.reshape(Q, _WAY)


# speedup vs baseline: 75.4791x; 1.8404x over previous
import jax
import jax.numpy as jnp
from jax import lax
from jax.experimental import pallas as pl
from jax.experimental.pallas import tpu as pltpu

_WAY = 5
_SHOT = 5
_K = 3
_QB = 5


def _top3sum(x):
    n = x.shape[1]
    iota = lax.broadcasted_iota(jnp.int32, x.shape, 1)
    total = jnp.zeros((x.shape[0], 1), jnp.float32)
    for k in range(_K):
        m = jnp.max(x, axis=1, keepdims=True)
        total = total + m
        if k < _K - 1:
            eq = x == m
            first = jnp.min(jnp.where(eq, iota, n), axis=1, keepdims=True)
            x = jnp.where(iota == first, -jnp.inf, x)
    return total


def _body(q_ref, s_ref, wsw_ref, par_ref, out_ref):
    C, U = s_ref.shape
    HW = q_ref.shape[1]
    SHW = _SHOT * HW
    ws_b = par_ref[0, 0]
    msn_w = par_ref[0, 1]
    msn_b = par_ref[0, 2]
    wm_w = par_ref[0, 3]
    wm_b = par_ref[0, 4]

    s = s_ref[...]
    sn = s * lax.rsqrt(jnp.sum(s * s, axis=0, keepdims=True))
    pw = jnp.zeros((C, 1), jnp.float32)
    for v in range(_WAY):
        pv = jnp.mean(sn[:, v * SHW:(v + 1) * SHW], axis=1, keepdims=True)
        pw = pw + wsw_ref[0, v] * pv
    ssw = jax.nn.sigmoid(jnp.sum(pw * sn, axis=0, keepdims=True) + ws_b)

    R = _QB * HW
    q = q_ref[...].reshape(R, C)
    qn = q * lax.rsqrt(jnp.sum(q * q, axis=1, keepdims=True))

    outs = []
    for v in range(_WAY):
        Av = jnp.dot(qn, sn[:, v * SHW:(v + 1) * SHW],
                     preferred_element_type=jnp.float32)
        M = Av[:, 0:HW]
        for b in range(1, _SHOT):
            M = jnp.maximum(M, Av[:, b * HW:(b + 1) * HW])
        rel = _top3sum(msn_w * M + msn_b)
        w = jax.nn.sigmoid(wm_w * rel + wm_b)
        r2 = _top3sum(Av * ssw[:, v * SHW:(v + 1) * SHW])
        pq = jnp.sum((w * r2).reshape(_QB, HW), axis=1)
        outs.append(pq.reshape(_QB, 1, 1))
    out_ref[...] = jnp.concatenate(outs, axis=2)


@jax.jit
def kernel(x1, x2, ws_w, ws_b, msn_w, msn_b, wm_w, wm_b):
    Q, C, H, W = x1.shape
    HW = H * W
    S = x2.shape[0]
    U = S * HW

    x1t = x1.reshape(Q, C, HW).transpose(0, 2, 1)
    s_cols = x2.reshape(S, C, HW).transpose(1, 0, 2).reshape(C, U)
    s_perm = (s_cols.reshape(C, _WAY, HW, _SHOT)
              .transpose(0, 1, 3, 2).reshape(C, U))
    wsw = jnp.asarray(ws_w, jnp.float32).reshape(1, _WAY)
    par = jnp.stack([ws_b, msn_w, msn_b, wm_w, wm_b]).astype(
        jnp.float32).reshape(1, 5)

    out = pl.pallas_call(
        _body,
        grid=(Q // _QB,),
        in_specs=[
            pl.BlockSpec((_QB, HW, C), lambda i: (i, 0, 0)),
            pl.BlockSpec((C, U), lambda i: (0, 0)),
            pl.BlockSpec((1, _WAY), lambda i: (0, 0)),
            pl.BlockSpec((1, 5), lambda i: (0, 0)),
        ],
        out_specs=pl.BlockSpec((_QB, 1, _WAY), lambda i: (i, 0, 0)),
        out_shape=jax.ShapeDtypeStruct((Q, 1, _WAY), jnp.float32),
        compiler_params=pltpu.CompilerParams(
            dimension_semantics=("arbitrary",)),
    )(x1t, s_perm, wsw, par)
    return out.reshape(Q, _WAY)


# speedup vs baseline: 75.5393x; 1.0008x over previous
import jax
import jax.numpy as jnp
from jax import lax
from jax.experimental import pallas as pl
from jax.experimental.pallas import tpu as pltpu

_WAY = 5
_SHOT = 5
_K = 3
_QB = 5


def _top3sum(x):
    n = x.shape[1]
    iota = lax.broadcasted_iota(jnp.int32, x.shape, 1)
    total = jnp.zeros((x.shape[0], 1), jnp.float32)
    for k in range(_K):
        m = jnp.max(x, axis=1, keepdims=True)
        total = total + m
        if k < _K - 1:
            eq = x == m
            first = jnp.min(jnp.where(eq, iota, n), axis=1, keepdims=True)
            x = jnp.where(iota == first, -jnp.inf, x)
    return total


def _body(q_ref, s_ref, wsw_ref, par_ref, out_ref):
    C, U = s_ref.shape
    HW = q_ref.shape[1]
    SHW = _SHOT * HW
    ws_b = par_ref[0, 0]
    msn_w = par_ref[0, 1]
    msn_b = par_ref[0, 2]
    wm_w = par_ref[0, 3]
    wm_b = par_ref[0, 4]

    s = s_ref[...]
    sn = s * lax.rsqrt(jnp.sum(s * s, axis=0, keepdims=True))
    pw = jnp.zeros((C, 1), jnp.float32)
    for v in range(_WAY):
        pv = jnp.mean(sn[:, v * SHW:(v + 1) * SHW], axis=1, keepdims=True)
        pw = pw + wsw_ref[0, v] * pv
    ssw = jax.nn.sigmoid(jnp.sum(pw * sn, axis=0, keepdims=True) + ws_b)

    R = _QB * HW
    q = q_ref[...].reshape(R, C)
    qn = q * lax.rsqrt(jnp.sum(q * q, axis=1, keepdims=True))

    outs = []
    for v in range(_WAY):
        Av = jnp.dot(qn, sn[:, v * SHW:(v + 1) * SHW],
                     preferred_element_type=jnp.float32)
        M = Av[:, 0:HW]
        for b in range(1, _SHOT):
            M = jnp.maximum(M, Av[:, b * HW:(b + 1) * HW])
        rel = _top3sum(msn_w * M + msn_b)
        w = jax.nn.sigmoid(wm_w * rel + wm_b)
        r2 = _top3sum(Av * ssw[:, v * SHW:(v + 1) * SHW])
        pq = jnp.sum((w * r2).reshape(_QB, HW), axis=1)
        outs.append(pq.reshape(_QB, 1, 1))
    out_ref[...] = jnp.concatenate(outs, axis=2)


@jax.jit
def kernel(x1, x2, ws_w, ws_b, msn_w, msn_b, wm_w, wm_b):
    Q, C, H, W = x1.shape
    HW = H * W
    S = x2.shape[0]
    U = S * HW

    x1t = x1.reshape(Q, C, HW).transpose(0, 2, 1)
    s_cols = x2.reshape(S, C, HW).transpose(1, 0, 2).reshape(C, U)
    s_perm = (s_cols.reshape(C, _WAY, HW, _SHOT)
              .transpose(0, 1, 3, 2).reshape(C, U))
    wsw = jnp.asarray(ws_w, jnp.float32).reshape(1, _WAY)
    par = jnp.stack([ws_b, msn_w, msn_b, wm_w, wm_b]).astype(
        jnp.float32).reshape(1, 5)

    out = pl.pallas_call(
        _body,
        grid=(Q // _QB,),
        in_specs=[
            pl.BlockSpec((_QB, HW, C), lambda i: (i, 0, 0)),
            pl.BlockSpec((C, U), lambda i: (0, 0)),
            pl.BlockSpec((1, _WAY), lambda i: (0, 0)),
            pl.BlockSpec((1, 5), lambda i: (0, 0)),
        ],
        out_specs=pl.BlockSpec((_QB, 1, _WAY), lambda i: (i, 0, 0)),
        out_shape=jax.ShapeDtypeStruct((Q, 1, _WAY), jnp.float32),
        compiler_params=pltpu.CompilerParams(
            dimension_semantics=("parallel",)),
    )(x1t, s_perm, wsw, par)
    return out.reshape(Q, _WAY)


# speedup vs baseline: 82.6426x; 1.0940x over previous
import jax
import jax.numpy as jnp
from jax import lax
from jax.experimental import pallas as pl
from jax.experimental.pallas import tpu as pltpu

_WAY = 5
_SHOT = 5
_K = 3
_QB = 5


def _top3sum(x):
    m1 = jnp.max(x, axis=1, keepdims=True)
    eq1 = x == m1
    c1 = jnp.sum(eq1.astype(jnp.float32), axis=1, keepdims=True)
    x2 = jnp.where(eq1, -jnp.inf, x)
    m2 = jnp.max(x2, axis=1, keepdims=True)
    eq2 = x2 == m2
    c2 = jnp.sum(eq2.astype(jnp.float32), axis=1, keepdims=True)
    x3 = jnp.where(eq2, -jnp.inf, x2)
    m3 = jnp.max(x3, axis=1, keepdims=True)
    k1 = jnp.minimum(c1, 3.0)
    k2 = jnp.minimum(c2, 3.0 - k1)
    k3 = jnp.maximum(3.0 - k1 - k2, 0.0)
    t = m1 * k1
    t = t + jnp.where(k2 > 0, m2 * k2, 0.0)
    t = t + jnp.where(k3 > 0, m3 * k3, 0.0)
    return t


def _body(q_ref, s_ref, wsw_ref, par_ref, out_ref):
    C, U = s_ref.shape
    HW = q_ref.shape[1]
    SHW = _SHOT * HW
    ws_b = par_ref[0, 0]
    msn_w = par_ref[0, 1]
    msn_b = par_ref[0, 2]
    wm_w = par_ref[0, 3]
    wm_b = par_ref[0, 4]

    s = s_ref[...]
    sn = s * lax.rsqrt(jnp.sum(s * s, axis=0, keepdims=True))
    pw = jnp.zeros((C, 1), jnp.float32)
    for v in range(_WAY):
        pv = jnp.mean(sn[:, v * SHW:(v + 1) * SHW], axis=1, keepdims=True)
        pw = pw + wsw_ref[0, v] * pv
    ssw = jax.nn.sigmoid(jnp.sum(pw * sn, axis=0, keepdims=True) + ws_b)

    R = _QB * HW
    q = q_ref[...].reshape(R, C)
    qn = q * lax.rsqrt(jnp.sum(q * q, axis=1, keepdims=True))

    outs = []
    for v in range(_WAY):
        Av = jnp.dot(qn, sn[:, v * SHW:(v + 1) * SHW],
                     preferred_element_type=jnp.float32)
        M = Av[:, 0:HW]
        for b in range(1, _SHOT):
            M = jnp.maximum(M, Av[:, b * HW:(b + 1) * HW])
        rel = _top3sum(msn_w * M + msn_b)
        w = jax.nn.sigmoid(wm_w * rel + wm_b)
        r2 = _top3sum(Av * ssw[:, v * SHW:(v + 1) * SHW])
        pq = jnp.sum((w * r2).reshape(_QB, HW), axis=1)
        outs.append(pq.reshape(_QB, 1, 1))
    out_ref[...] = jnp.concatenate(outs, axis=2)


@jax.jit
def kernel(x1, x2, ws_w, ws_b, msn_w, msn_b, wm_w, wm_b):
    Q, C, H, W = x1.shape
    HW = H * W
    S = x2.shape[0]
    U = S * HW

    x1t = x1.reshape(Q, C, HW).transpose(0, 2, 1)
    s_cols = x2.reshape(S, C, HW).transpose(1, 0, 2).reshape(C, U)
    s_perm = (s_cols.reshape(C, _WAY, HW, _SHOT)
              .transpose(0, 1, 3, 2).reshape(C, U))
    wsw = jnp.asarray(ws_w, jnp.float32).reshape(1, _WAY)
    par = jnp.stack([ws_b, msn_w, msn_b, wm_w, wm_b]).astype(
        jnp.float32).reshape(1, 5)

    out = pl.pallas_call(
        _body,
        grid=(Q // _QB,),
        in_specs=[
            pl.BlockSpec((_QB, HW, C), lambda i: (i, 0, 0)),
            pl.BlockSpec((C, U), lambda i: (0, 0)),
            pl.BlockSpec((1, _WAY), lambda i: (0, 0)),
            pl.BlockSpec((1, 5), lambda i: (0, 0)),
        ],
        out_specs=pl.BlockSpec((_QB, 1, _WAY), lambda i: (i, 0, 0)),
        out_shape=jax.ShapeDtypeStruct((Q, 1, _WAY), jnp.float32),
        compiler_params=pltpu.CompilerParams(
            dimension_semantics=("parallel",)),
    )(x1t, s_perm, wsw, par)
    return out.reshape(Q, _WAY)
